# Initial kernel scaffold; baseline (speedup 1.0000x reference)
import jax
import jax.numpy as jnp
from jax.experimental import pallas as pl


def kernel(X, L, batch, W1, b1, W2, b2, W3, b3, W4, b4):
    raise NotImplementedError("write your pallas kernel here")


# speedup vs baseline: 1.1992x; 1.1992x over previous
import functools

import jax
import jax.numpy as jnp
from jax.experimental import pallas as pl
from jax.experimental.pallas import tpu as pltpu

N = 10000
D = 128
H = 64
OUT = 32
G = 8
BM = 400
GRID = N // BM

_f32 = jnp.float32


def _pass1_body(x0_ref, w_ref, b_ref, l_ref, out_ref, ybf_ref, ysum_ref):
    @pl.when(pl.program_id(0) == 0)
    def _():
        y = jnp.dot(x0_ref[...], w_ref[...], preferred_element_type=_f32)
        y = y + b_ref[...]
        ybf_ref[...] = y
        ysum_ref[...] = jnp.sum(y, axis=0, keepdims=True)

    mm = jnp.dot(l_ref[...], ybf_ref[...], preferred_element_type=_f32)
    out_ref[...] = jnp.maximum(mm, 0.0)


def _pass3_body(xprev_ref, w_ref, b_ref, l_ref, x1_ref, x2_ref, avg_ref,
                ybf_ref, ysum_ref):
    @pl.when(pl.program_id(0) == 0)
    def _():
        y = jnp.dot(xprev_ref[...], w_ref[...], preferred_element_type=_f32)
        y = y + b_ref[...]
        ybf_ref[...] = y
        ysum_ref[...] = jnp.sum(y, axis=0, keepdims=True)

    mm = jnp.dot(l_ref[...], ybf_ref[...], preferred_element_type=_f32)
    x3 = jnp.maximum(mm, 0.0)
    avg_ref[...] = (x1_ref[...] + x2_ref[...] + x3) * (1.0 / 3.0)


def _gcn_pass(xprev, L0, W, b):
    din = xprev.shape[1]
    return pl.pallas_call(
        _pass1_body,
        grid=(GRID,),
        in_specs=[
            pl.BlockSpec((N, din), lambda i: (0, 0)),
            pl.BlockSpec((din, H), lambda i: (0, 0)),
            pl.BlockSpec((1, H), lambda i: (0, 0)),
            pl.BlockSpec((BM, N), lambda i: (i, 0)),
        ],
        out_specs=pl.BlockSpec((BM, H), lambda i: (i, 0)),
        out_shape=jax.ShapeDtypeStruct((N, H), _f32),
        scratch_shapes=[
            pltpu.VMEM((N, H), _f32),
            pltpu.VMEM((1, H), _f32),
        ],
    )(xprev, W, b.reshape(1, H), L0)


def _gcn_pass3(xprev, L0, W, b, x1, x2):
    return pl.pallas_call(
        _pass3_body,
        grid=(GRID,),
        in_specs=[
            pl.BlockSpec((N, H), lambda i: (0, 0)),
            pl.BlockSpec((H, H), lambda i: (0, 0)),
            pl.BlockSpec((1, H), lambda i: (0, 0)),
            pl.BlockSpec((BM, N), lambda i: (i, 0)),
            pl.BlockSpec((BM, H), lambda i: (i, 0)),
            pl.BlockSpec((BM, H), lambda i: (i, 0)),
        ],
        out_specs=pl.BlockSpec((BM, H), lambda i: (i, 0)),
        out_shape=jax.ShapeDtypeStruct((N, H), _f32),
        scratch_shapes=[
            pltpu.VMEM((N, H), _f32),
            pltpu.VMEM((1, H), _f32),
        ],
    )(xprev, W, b.reshape(1, H), L0, x1, x2)


def _head_body(avg_ref, ids_ref, w4_ref, b4_ref, out_ref):
    ids = ids_ref[...]
    seg = jax.lax.broadcasted_iota(jnp.int32, (G, N), 0)
    onehot = (ids == seg).astype(_f32)
    sums = jnp.dot(onehot, avg_ref[...], preferred_element_type=_f32)
    counts = jnp.sum(onehot, axis=1, keepdims=True)
    pooled = sums / jnp.maximum(counts, 1.0)
    logits = jnp.dot(pooled, w4_ref[...], preferred_element_type=_f32)
    logits = logits + b4_ref[...]
    m = jnp.max(logits, axis=1, keepdims=True)
    e = jnp.exp(logits - m)
    out_ref[...] = e / jnp.sum(e, axis=1, keepdims=True)


def _head(avg, ids, W4, b4):
    return pl.pallas_call(
        _head_body,
        in_specs=[
            pl.BlockSpec((N, H), lambda: (0, 0)),
            pl.BlockSpec((1, N), lambda: (0, 0)),
            pl.BlockSpec((H, OUT), lambda: (0, 0)),
            pl.BlockSpec((1, OUT), lambda: (0, 0)),
        ],
        out_specs=pl.BlockSpec((G, OUT), lambda: (0, 0)),
        out_shape=jax.ShapeDtypeStruct((G, OUT), _f32),
    )(avg, ids.reshape(1, N), W4, b4.reshape(1, OUT))


def kernel(X, L, batch, W1, b1, W2, b2, W3, b3, W4, b4):
    X0 = X[0]
    L0 = L[0]
    ids = batch[0].astype(jnp.int32)
    x1 = _gcn_pass(X0, L0, W1, b1)
    x2 = _gcn_pass(x1, L0, W2, b2)
    avg = _gcn_pass3(x2, L0, W3, b3, x1, x2)
    return _head(avg, ids, W4, b4)


# speedup vs baseline: 1.4133x; 1.1786x over previous
import jax
import jax.numpy as jnp
from jax.experimental import pallas as pl
from jax.experimental.pallas import tpu as pltpu

N = 10000
D = 128
H = 64
OUT = 32
G = 8
BM1 = 256
BM2 = 512

_f32 = jnp.float32
_bf16 = jnp.bfloat16


def _pass1_body(x0_ref, w_ref, b_ref, l_ref, x1_ref, lq_ref, ac_ref,
                ybf_ref, ysum_ref):
    @pl.when(pl.program_id(0) == 0)
    def _():
        y = jnp.dot(x0_ref[...], w_ref[...], preferred_element_type=_f32)
        y = y + b_ref[...]
        ybf_ref[...] = y.astype(_bf16)
        ysum_ref[...] = jnp.sum(y, axis=0, keepdims=True)

    l = l_ref[...]
    lo = jnp.min(l, axis=1, keepdims=True)
    hi = jnp.max(l, axis=1, keepdims=True)
    rng = jnp.maximum(hi - lo, 1e-30)
    step = rng * (1.0 / 254.0)
    q = jnp.floor((l - lo) * (254.0 / rng) + 0.5) - 127.0
    lq_ref[...] = q.astype(jnp.int8)
    a = step
    c = lo + 127.0 * step
    ac_ref[...] = jnp.concatenate([a, c], axis=1)
    mm = jnp.dot(q.astype(_bf16), ybf_ref[...], preferred_element_type=_f32)
    x1_ref[...] = jnp.maximum(a * mm + c * ysum_ref[...], 0.0)


def _pass2_body(xprev_ref, w_ref, b_ref, lq_ref, ac_ref, out_ref,
                ybf_ref, ysum_ref):
    @pl.when(pl.program_id(0) == 0)
    def _():
        y = jnp.dot(xprev_ref[...], w_ref[...], preferred_element_type=_f32)
        y = y + b_ref[...]
        ybf_ref[...] = y.astype(_bf16)
        ysum_ref[...] = jnp.sum(y, axis=0, keepdims=True)

    q = lq_ref[...].astype(_bf16)
    mm = jnp.dot(q, ybf_ref[...], preferred_element_type=_f32)
    a = ac_ref[:, 0:1]
    c = ac_ref[:, 1:2]
    out_ref[...] = jnp.maximum(a * mm + c * ysum_ref[...], 0.0)


def _pass3_body(xprev_ref, w_ref, b_ref, lq_ref, ac_ref, x1_ref, x2_ref,
                avg_ref, ybf_ref, ysum_ref):
    @pl.when(pl.program_id(0) == 0)
    def _():
        y = jnp.dot(xprev_ref[...], w_ref[...], preferred_element_type=_f32)
        y = y + b_ref[...]
        ybf_ref[...] = y.astype(_bf16)
        ysum_ref[...] = jnp.sum(y, axis=0, keepdims=True)

    q = lq_ref[...].astype(_bf16)
    mm = jnp.dot(q, ybf_ref[...], preferred_element_type=_f32)
    a = ac_ref[:, 0:1]
    c = ac_ref[:, 1:2]
    x3 = jnp.maximum(a * mm + c * ysum_ref[...], 0.0)
    avg_ref[...] = (x1_ref[...] + x2_ref[...] + x3) * (1.0 / 3.0)


def _gcn_pass1(x0, L0, W, b):
    grid1 = pl.cdiv(N, BM1)
    return pl.pallas_call(
        _pass1_body,
        grid=(grid1,),
        in_specs=[
            pl.BlockSpec((N, D), lambda i: (0, 0)),
            pl.BlockSpec((D, H), lambda i: (0, 0)),
            pl.BlockSpec((1, H), lambda i: (0, 0)),
            pl.BlockSpec((BM1, N), lambda i: (i, 0)),
        ],
        out_specs=[
            pl.BlockSpec((BM1, H), lambda i: (i, 0)),
            pl.BlockSpec((BM1, N), lambda i: (i, 0)),
            pl.BlockSpec((BM1, 2), lambda i: (i, 0)),
        ],
        out_shape=[
            jax.ShapeDtypeStruct((N, H), _f32),
            jax.ShapeDtypeStruct((N, N), jnp.int8),
            jax.ShapeDtypeStruct((N, 2), _f32),
        ],
        scratch_shapes=[
            pltpu.VMEM((N, H), _bf16),
            pltpu.VMEM((1, H), _f32),
        ],
    )(x0, W, b.reshape(1, H), L0)


def _gcn_pass2(xprev, lq, ac, W, b):
    grid2 = pl.cdiv(N, BM2)
    return pl.pallas_call(
        _pass2_body,
        grid=(grid2,),
        in_specs=[
            pl.BlockSpec((N, H), lambda i: (0, 0)),
            pl.BlockSpec((H, H), lambda i: (0, 0)),
            pl.BlockSpec((1, H), lambda i: (0, 0)),
            pl.BlockSpec((BM2, N), lambda i: (i, 0)),
            pl.BlockSpec((BM2, 2), lambda i: (i, 0)),
        ],
        out_specs=pl.BlockSpec((BM2, H), lambda i: (i, 0)),
        out_shape=jax.ShapeDtypeStruct((N, H), _f32),
        scratch_shapes=[
            pltpu.VMEM((N, H), _bf16),
            pltpu.VMEM((1, H), _f32),
        ],
    )(xprev, W, b.reshape(1, H), lq, ac)


def _gcn_pass3(xprev, lq, ac, W, b, x1, x2):
    grid2 = pl.cdiv(N, BM2)
    return pl.pallas_call(
        _pass3_body,
        grid=(grid2,),
        in_specs=[
            pl.BlockSpec((N, H), lambda i: (0, 0)),
            pl.BlockSpec((H, H), lambda i: (0, 0)),
            pl.BlockSpec((1, H), lambda i: (0, 0)),
            pl.BlockSpec((BM2, N), lambda i: (i, 0)),
            pl.BlockSpec((BM2, 2), lambda i: (i, 0)),
            pl.BlockSpec((BM2, H), lambda i: (i, 0)),
            pl.BlockSpec((BM2, H), lambda i: (i, 0)),
        ],
        out_specs=pl.BlockSpec((BM2, H), lambda i: (i, 0)),
        out_shape=jax.ShapeDtypeStruct((N, H), _f32),
        scratch_shapes=[
            pltpu.VMEM((N, H), _bf16),
            pltpu.VMEM((1, H), _f32),
        ],
    )(xprev, W, b.reshape(1, H), lq, ac, x1, x2)


def _head_body(avg_ref, ids_ref, w4_ref, b4_ref, out_ref):
    ids = ids_ref[...]
    seg = jax.lax.broadcasted_iota(jnp.int32, (G, N), 0)
    onehot = (ids == seg).astype(_f32)
    sums = jnp.dot(onehot, avg_ref[...], preferred_element_type=_f32)
    counts = jnp.sum(onehot, axis=1, keepdims=True)
    pooled = sums / jnp.maximum(counts, 1.0)
    logits = jnp.dot(pooled, w4_ref[...], preferred_element_type=_f32)
    logits = logits + b4_ref[...]
    m = jnp.max(logits, axis=1, keepdims=True)
    e = jnp.exp(logits - m)
    out_ref[...] = e / jnp.sum(e, axis=1, keepdims=True)


def _head(avg, ids, W4, b4):
    return pl.pallas_call(
        _head_body,
        in_specs=[
            pl.BlockSpec((N, H), lambda: (0, 0)),
            pl.BlockSpec((1, N), lambda: (0, 0)),
            pl.BlockSpec((H, OUT), lambda: (0, 0)),
            pl.BlockSpec((1, OUT), lambda: (0, 0)),
        ],
        out_specs=pl.BlockSpec((G, OUT), lambda: (0, 0)),
        out_shape=jax.ShapeDtypeStruct((G, OUT), _f32),
    )(avg, ids.reshape(1, N), W4, b4.reshape(1, OUT))


def kernel(X, L, batch, W1, b1, W2, b2, W3, b3, W4, b4):
    X0 = X[0]
    L0 = L[0]
    ids = batch[0].astype(jnp.int32)
    x1, lq, ac = _gcn_pass1(X0, L0, W1, b1)
    x2 = _gcn_pass2(x1, lq, ac, W2, b2)
    avg = _gcn_pass3(x2, lq, ac, W3, b3, x1, x2)
    return _head(avg, ids, W4, b4)
